# baseline (speedup 1.0000x reference)
import functools

import jax
import jax.numpy as jnp
from jax import lax
from jax.experimental import pallas as pl
from jax.experimental.pallas import tpu as pltpu
from jax.experimental.pallas import tpu_sc as plsc

_B, _T, _N, _F = 32, 12, 1024, 128
_D = 288
_W = 7
_HF = _F // 2
_L = 16
_NC = 256
_NCHUNKS = _T * (_N // _NC)
_DS = 65
_WS = 449
_TN = _T * _N


def _emb_body(xday_hbm, xweek_hbm, dayt_hbm, weekt_hbm, out_hbm,
              xd_v, xw_v, day_v, week_v, ob0, ob1, sem0, sem1):
    b = lax.axis_index("s") * 2 + lax.axis_index("c")

    pltpu.async_copy(dayt_hbm, day_v, sem0)
    pltpu.async_copy(weekt_hbm, week_v, sem0)
    pltpu.async_copy(xday_hbm.at[pl.ds(b * _TN, _TN)], xd_v, sem0)
    pltpu.async_copy(xweek_hbm.at[pl.ds(b * _TN, _TN)], xw_v, sem0)
    pltpu.make_async_copy(dayt_hbm, day_v, sem0).wait()
    pltpu.make_async_copy(weekt_hbm, week_v, sem0).wait()
    pltpu.make_async_copy(xday_hbm.at[pl.ds(b * _TN, _TN)], xd_v, sem0).wait()
    pltpu.make_async_copy(xweek_hbm.at[pl.ds(b * _TN, _TN)], xw_v, sem0).wait()

    lanes = lax.iota(jnp.int32, _L)
    lane_ws = lanes * _WS

    def fill(ob, t, n0):
        base = t * _N + n0

        def fill_g(g, _):
            xdv = xd_v[pl.ds(base + g * _L, _L)]
            xwv = xw_v[pl.ds(base + g * _L, _L)]
            dv0 = (xdv * jnp.float32(_D)).astype(jnp.int32) * _DS
            wv0 = xwv.astype(jnp.int32) * _HF + lane_ws
            col = g * _L
            ch = col // 128
            cl = col % 128

            @plsc.parallel_loop(0, _HF, 1, unroll=16)
            def _fill_f(f):
                fa = lax.shift_right_logical(f, 3)
                fb = lax.rem(f, 8)
                ob[fa, ch, fb, pl.ds(cl, _L)] = plsc.load_gather(
                    day_v, [dv0 + f])
                ob[fa + _HF // 8, ch, fb, pl.ds(cl, _L)] = plsc.load_gather(
                    week_v, [wv0 + f])

            return 0

        lax.fori_loop(0, _NC // _L, fill_g, 0)

    _NT = _NC // 128

    def pair_body(cp, _):
        for half, (ob, sem) in enumerate(((ob0, sem0), (ob1, sem1))):
            ci = cp * 2 + half
            t = lax.div(ci, _N // _NC)
            n0 = lax.rem(ci, _N // _NC) * _NC
            na0 = lax.div(n0, 128)

            @pl.when(cp > 0)
            def _wait_prev():
                pltpu.make_async_copy(
                    ob, out_hbm.at[b, 0, :, pl.ds(0, _NT), :, :], sem).wait()

            fill(ob, t, n0)
            pltpu.async_copy(
                ob, out_hbm.at[b, t, :, pl.ds(na0, _NT), :, :], sem)
        return 0

    lax.fori_loop(0, _NCHUNKS // 2, pair_body, 0)
    pltpu.make_async_copy(
        ob0, out_hbm.at[b, 0, :, pl.ds(0, _NT), :, :], sem0).wait()
    pltpu.make_async_copy(
        ob1, out_hbm.at[b, 0, :, pl.ds(0, _NT), :, :], sem1).wait()


_emb = functools.partial(
    pl.kernel,
    mesh=plsc.VectorSubcoreMesh(core_axis_name="c", subcore_axis_name="s"),
    out_type=jax.ShapeDtypeStruct((_B, _T, _F // 8, _N // 128, 8, 128),
                                  jnp.float32),
    compiler_params=pltpu.CompilerParams(
        use_tc_tiling_on_sc=False, needs_layout_passes=False),
    scratch_types=[
        pltpu.VMEM((_TN,), jnp.float32),
        pltpu.VMEM((_TN,), jnp.float32),
        pltpu.VMEM((_D * _DS,), jnp.float32),
        pltpu.VMEM((_L * _WS,), jnp.float32),
        pltpu.VMEM((_F // 8, _NC // 128, 8, 128), jnp.float32),
        pltpu.VMEM((_F // 8, _NC // 128, 8, 128), jnp.float32),
        pltpu.SemaphoreType.DMA,
        pltpu.SemaphoreType.DMA,
    ],
)(_emb_body)


def kernel(x, time_day_table, time_week_table):
    xday = x[:, :, :, 1].reshape(-1)
    xweek = x[:, :, :, 2].reshape(-1)
    day_pad = jnp.pad(time_day_table, ((0, 0), (0, _DS - _HF))).reshape(-1)
    week_rep = jnp.pad(
        jnp.tile(time_week_table.reshape(1, _W * _HF), (_L, 1)),
        ((0, 0), (0, _WS - _W * _HF))).reshape(-1)
    out6 = _emb(xday, xweek, day_pad, week_rep)
    return jnp.transpose(out6, (0, 2, 4, 3, 5, 1)).reshape(_B, _F, _N, _T)


# speedup vs baseline: 1.0030x; 1.0030x over previous
import functools

import jax
import jax.numpy as jnp
from jax import lax
from jax.experimental import pallas as pl
from jax.experimental.pallas import tpu as pltpu
from jax.experimental.pallas import tpu_sc as plsc

_B, _T, _N, _F = 32, 12, 1024, 128
_D = 288
_W = 7
_HF = _F // 2
_L = 16
_NC = 256
_NCHUNKS = _T * (_N // _NC)
_DS = 65
_WS = 449
_TN = _T * _N


def _emb_body(xday_hbm, xweek_hbm, dayt_hbm, weekt_hbm, out_hbm,
              xd_v, xw_v, day_v, week_v, ob0, ob1, sem0, sem1):
    b = lax.axis_index("s") * 2 + lax.axis_index("c")

    pltpu.sync_copy(dayt_hbm, day_v)
    pltpu.sync_copy(weekt_hbm, week_v)
    pltpu.sync_copy(xday_hbm.at[pl.ds(b * _TN, _TN)], xd_v)
    pltpu.sync_copy(xweek_hbm.at[pl.ds(b * _TN, _TN)], xw_v)

    lanes = lax.iota(jnp.int32, _L)
    lane_ws = lanes * _WS

    def fill(ob, t, n0):
        base = t * _N + n0

        def fill_g(g, _):
            xdv = xd_v[pl.ds(base + g * _L, _L)]
            xwv = xw_v[pl.ds(base + g * _L, _L)]
            dv0 = (xdv * jnp.float32(_D)).astype(jnp.int32) * _DS
            wv0 = xwv.astype(jnp.int32) * _HF + lane_ws
            col = g * _L
            ch = col // 128
            cl = col % 128

            @plsc.parallel_loop(0, _HF, 1, unroll=8)
            def _fill_f(f):
                fa = lax.shift_right_logical(f, 3)
                fb = lax.rem(f, 8)
                ob[fa, ch, fb, pl.ds(cl, _L)] = plsc.load_gather(
                    day_v, [dv0 + f])
                ob[fa + _HF // 8, ch, fb, pl.ds(cl, _L)] = plsc.load_gather(
                    week_v, [wv0 + f])

            return 0

        lax.fori_loop(0, _NC // _L, fill_g, 0)

    _NT = _NC // 128

    def pair_body(cp, _):
        for half, (ob, sem) in enumerate(((ob0, sem0), (ob1, sem1))):
            ci = cp * 2 + half
            t = lax.div(ci, _N // _NC)
            n0 = lax.rem(ci, _N // _NC) * _NC
            na0 = lax.div(n0, 128)

            @pl.when(cp > 0)
            def _wait_prev():
                pltpu.make_async_copy(
                    ob, out_hbm.at[b, 0, :, pl.ds(0, _NT), :, :], sem).wait()

            fill(ob, t, n0)
            pltpu.async_copy(
                ob, out_hbm.at[b, t, :, pl.ds(na0, _NT), :, :], sem)
        return 0

    lax.fori_loop(0, _NCHUNKS // 2, pair_body, 0)
    pltpu.make_async_copy(
        ob0, out_hbm.at[b, 0, :, pl.ds(0, _NT), :, :], sem0).wait()
    pltpu.make_async_copy(
        ob1, out_hbm.at[b, 0, :, pl.ds(0, _NT), :, :], sem1).wait()


_emb = functools.partial(
    pl.kernel,
    mesh=plsc.VectorSubcoreMesh(core_axis_name="c", subcore_axis_name="s"),
    out_type=jax.ShapeDtypeStruct((_B, _T, _F // 8, _N // 128, 8, 128),
                                  jnp.float32),
    compiler_params=pltpu.CompilerParams(
        use_tc_tiling_on_sc=False, needs_layout_passes=False),
    scratch_types=[
        pltpu.VMEM((_TN,), jnp.float32),
        pltpu.VMEM((_TN,), jnp.float32),
        pltpu.VMEM((_D * _DS,), jnp.float32),
        pltpu.VMEM((_L * _WS,), jnp.float32),
        pltpu.VMEM((_F // 8, _NC // 128, 8, 128), jnp.float32),
        pltpu.VMEM((_F // 8, _NC // 128, 8, 128), jnp.float32),
        pltpu.SemaphoreType.DMA,
        pltpu.SemaphoreType.DMA,
    ],
)(_emb_body)


def kernel(x, time_day_table, time_week_table):
    xday = x[:, :, :, 1].reshape(-1)
    xweek = x[:, :, :, 2].reshape(-1)
    day_pad = jnp.pad(time_day_table, ((0, 0), (0, _DS - _HF))).reshape(-1)
    week_rep = jnp.pad(
        jnp.tile(time_week_table.reshape(1, _W * _HF), (_L, 1)),
        ((0, 0), (0, _WS - _W * _HF))).reshape(-1)
    out6 = _emb(xday, xweek, day_pad, week_rep)
    return jnp.transpose(out6, (0, 2, 4, 3, 5, 1)).reshape(_B, _F, _N, _T)
